# baseline (speedup 1.0000x reference)
import jax
import jax.numpy as jnp
from jax import lax
from jax.experimental import pallas as pl
from jax.experimental.pallas import tpu as pltpu
from jax.experimental.pallas import tpu_sc as plsc

NC = 2
NS = 16
NW = NC * NS
LANES = 16
BW = 128


def _make_sc_call(N, B, L, V, MJ, ND, ED, E):
    PER_W = N // NW
    assert PER_W == BW * L
    LCH = 5
    C = LCH * BW
    NCHUNK = L // LCH
    EW = ED - 3
    NB1 = B // BW
    NC1 = (ND + ED) // 8

    def body(*refs):
        (idx_hbm, jp_hbm, bb_hbm, emb_hbm, emapt_hbm, ewt_hbm) = refs[:6]
        btf_hbm, out_hbm = refs[6], refs[7]
        (idxl_v, jpl_v, bbl_v, glist_v, fidx_v, eidx_v, embr_v, embf_v,
         eet_v, btab_v, sem_in, sem_e, sem_m, sem_w, sem_o) = refs[8:]
        wid = lax.axis_index("s") * NC + lax.axis_index("c")
        lane = lax.iota(jnp.int32, LANES)
        wbase = wid * PER_W
        pltpu.sync_copy(btf_hbm, btab_v)
        pltpu.async_copy(idx_hbm.at[pl.ds(wbase, PER_W)], idxl_v, sem_in)
        pltpu.async_copy(jp_hbm.at[pl.ds(wbase, PER_W)], jpl_v, sem_in)
        pltpu.async_copy(bb_hbm.at[pl.ds(wbase, PER_W)], bbl_v, sem_in)
        for r in (idxl_v, jpl_v, bbl_v):
            pltpu.make_async_copy(idx_hbm.at[pl.ds(0, PER_W)], r, sem_in).wait()

        n_out_dma = LCH * NC1

        def do_chunk(ch, first):
            if not first:
                for _ in range(n_out_dma):
                    pltpu.make_async_copy(
                        embf_v.at[pl.ds(0, 8), pl.ds(0, BW)],
                        out_hbm.at[0, 0, wid], sem_o).wait()

            def list_body(i, c2):
                s = pl.ds(i * LANES, LANES)
                tau = lane + i * LANES
                tl = (tau & (BW - 1)) * L + (ch * LCH + (tau >> 7))
                gi = plsc.load_gather(idxl_v, [tl])
                glist_v[s] = gi
                fidx_v[s] = plsc.load_gather(jpl_v, [tl]) * V + gi
                bb16 = plsc.load_gather(bbl_v, [tl])
                for j in range(3):
                    eet_v[EW + j, s] = plsc.load_gather(btab_v, [bb16 * 3 + j])
                return c2

            lax.fori_loop(0, C // LANES, list_body, 0)
            cp_emb = pltpu.async_copy(emb_hbm.at[glist_v], embr_v, sem_e)
            pltpu.async_copy(emapt_hbm.at[fidx_v], eidx_v, sem_m).wait()

            def eidx_body(i, c2):
                s = pl.ds(i * LANES, LANES)
                eidx_v[s] = eidx_v[s] + 1
                return c2

            lax.fori_loop(0, C // LANES, eidx_body, 0)
            cps = [pltpu.async_copy(ewt_hbm.at[c].at[eidx_v], eet_v.at[c], sem_w)
                   for c in range(EW)]
            cp_emb.wait()

            def tr_body(tau, c2):
                for k in range(ND // LANES):
                    plsc.store_scatter(
                        embf_v,
                        [lane + k * LANES, jnp.full((LANES,), tau, jnp.int32)],
                        embr_v[tau, pl.ds(k * LANES, LANES)])
                return c2

            lax.fori_loop(0, C, tr_body, 0)
            for cp in cps:
                cp.wait()
            for lp in range(LCH):
                lg = ch * LCH + lp
                for c1 in range(NC1):
                    src = embf_v if c1 < ND // 8 else eet_v
                    r0 = c1 * 8 if c1 < ND // 8 else (c1 - ND // 8) * 8
                    pltpu.async_copy(
                        src.at[pl.ds(r0, 8), pl.ds(lp * BW, BW)],
                        out_hbm.at[lg, c1, wid], sem_o)
            return 0

        do_chunk(0, True)
        lax.fori_loop(1, NCHUNK, lambda ch, c: do_chunk(ch, False), 0)
        for _ in range(n_out_dma):
            pltpu.make_async_copy(
                embf_v.at[pl.ds(0, 8), pl.ds(0, BW)],
                out_hbm.at[0, 0, wid], sem_o).wait()

    return pl.kernel(
        body,
        out_type=jax.ShapeDtypeStruct((L, NC1, NB1, 8, BW), jnp.float32),
        mesh=plsc.VectorSubcoreMesh(core_axis_name="c", subcore_axis_name="s",
                                    num_cores=NC, num_subcores=NS),
        compiler_params=pltpu.CompilerParams(use_tc_tiling_on_sc=False,
                                             needs_layout_passes=False),
        scratch_types=[
            pltpu.VMEM((PER_W,), jnp.int32),
            pltpu.VMEM((PER_W,), jnp.int32),
            pltpu.VMEM((PER_W,), jnp.int32),
            pltpu.VMEM((C,), jnp.int32),
            pltpu.VMEM((C,), jnp.int32),
            pltpu.VMEM((C,), jnp.int32),
            pltpu.VMEM((C, ND), jnp.float32),
            pltpu.VMEM((ND, C), jnp.float32),
            pltpu.VMEM((16, C), jnp.float32),
            pltpu.VMEM((12,), jnp.float32),
            pltpu.SemaphoreType.DMA,
            pltpu.SemaphoreType.DMA,
            pltpu.SemaphoreType.DMA,
            pltpu.SemaphoreType.DMA,
            pltpu.SemaphoreType.DMA,
        ],
    )


def kernel(idx, joint_info, embedding, edge_idx_map, edge_emb_weight, bond_type):
    B, L = idx.shape
    N = B * L
    V, ND = embedding.shape
    MJ = edge_idx_map.shape[1]
    E, EW = edge_emb_weight.shape
    ED = EW + 3
    idx_f = idx.reshape(N)
    jp_f = joint_info[..., 0].reshape(N)
    bb_f = joint_info[..., 1].reshape(N)
    emap_t = edge_idx_map.T.reshape(MJ * V)
    ewt = edge_emb_weight.T
    bt_f = bond_type.reshape(-1)
    out5 = _make_sc_call(N, B, L, V, MJ, ND, ED, E)(
        idx_f, jp_f, bb_f, embedding, emap_t, ewt, bt_f)
    return out5.transpose(2, 4, 0, 1, 3).reshape(B, L, ND + ED)


# speedup vs baseline: 1.4797x; 1.4797x over previous
import jax
import jax.numpy as jnp
from jax import lax
from jax.experimental import pallas as pl
from jax.experimental.pallas import tpu as pltpu
from jax.experimental.pallas import tpu_sc as plsc

NC = 2
NS = 16
NW = NC * NS
LANES = 16
BW = 128


def _make_sc_call(N, B, L, V, MJ, ND, ED, E):
    PER_W = N // NW
    assert PER_W == BW * L
    LCH = 5
    C = LCH * BW
    NCHUNK = L // LCH
    EW = ED - 3
    NB1 = B // BW
    NC1 = (ND + ED) // 8

    def body(*refs):
        (idx_hbm, jp_hbm, bb_hbm, emb_hbm, emapt_hbm) = refs[:5]
        ewc_hbm = refs[5:5 + EW]
        btf_hbm, out_hbm = refs[5 + EW], refs[6 + EW]
        (idxl_v, jpl_v, bbl_v, glist_v, fidx_v, eidx_v, embr_v, embf_v,
         eet_v, btab_v, sem_in, sem_e, sem_m, sem_w, sem_o) = refs[7 + EW:]
        wid = lax.axis_index("s") * NC + lax.axis_index("c")
        lane = lax.iota(jnp.int32, LANES)
        wbase = wid * PER_W
        pltpu.sync_copy(btf_hbm, btab_v)
        pltpu.async_copy(idx_hbm.at[pl.ds(wbase, PER_W)], idxl_v, sem_in)
        pltpu.async_copy(jp_hbm.at[pl.ds(wbase, PER_W)], jpl_v, sem_in)
        pltpu.async_copy(bb_hbm.at[pl.ds(wbase, PER_W)], bbl_v, sem_in)
        for r in (idxl_v, jpl_v, bbl_v):
            pltpu.make_async_copy(idx_hbm.at[pl.ds(0, PER_W)], r, sem_in).wait()

        n_out_dma = LCH * NC1

        def do_chunk(ch, first):
            def list_body(i, c2):
                s = pl.ds(i * LANES, LANES)
                tau = lane + i * LANES
                tl = (tau & (BW - 1)) * L + (ch * LCH + (tau >> 7))
                gi = plsc.load_gather(idxl_v, [tl])
                glist_v[s] = gi
                fidx_v[s] = plsc.load_gather(jpl_v, [tl]) * V + gi
                return c2

            lax.fori_loop(0, C // LANES, list_body, 0)
            cp_emb = pltpu.async_copy(emb_hbm.at[glist_v], embr_v, sem_e)
            cp_map = pltpu.async_copy(emapt_hbm.at[fidx_v], eidx_v, sem_m)

            if not first:
                for _ in range(n_out_dma):
                    pltpu.make_async_copy(
                        embf_v.at[pl.ds(0, 8), pl.ds(0, BW)],
                        out_hbm.at[0, 0, wid], sem_o).wait()

            def bond_body(i, c2):
                s = pl.ds(i * LANES, LANES)
                tau = lane + i * LANES
                tl = (tau & (BW - 1)) * L + (ch * LCH + (tau >> 7))
                bb16 = plsc.load_gather(bbl_v, [tl])
                for j in range(3):
                    eet_v[EW + j, s] = plsc.load_gather(btab_v, [bb16 * 3 + j])
                return c2

            lax.fori_loop(0, C // LANES, bond_body, 0)
            cp_map.wait()

            def eidx_body(i, c2):
                s = pl.ds(i * LANES, LANES)
                eidx_v[s] = eidx_v[s] + 1
                return c2

            lax.fori_loop(0, C // LANES, eidx_body, 0)
            cps = [pltpu.async_copy(ewc_hbm[c].at[eidx_v], eet_v.at[c], sem_w)
                   for c in range(EW)]
            cp_emb.wait()

            def tr_body(tau, c2):
                for k in range(ND // LANES):
                    plsc.store_scatter(
                        embf_v,
                        [lane + k * LANES, jnp.full((LANES,), tau, jnp.int32)],
                        embr_v[tau, pl.ds(k * LANES, LANES)])
                return c2

            lax.fori_loop(0, C, tr_body, 0)
            for cp in cps:
                cp.wait()
            for lp in range(LCH):
                lg = ch * LCH + lp
                for c1 in range(NC1):
                    src = embf_v if c1 < ND // 8 else eet_v
                    r0 = c1 * 8 if c1 < ND // 8 else (c1 - ND // 8) * 8
                    pltpu.async_copy(
                        src.at[pl.ds(r0, 8), pl.ds(lp * BW, BW)],
                        out_hbm.at[lg, c1, wid], sem_o)
            return 0

        do_chunk(0, True)
        lax.fori_loop(1, NCHUNK, lambda ch, c: do_chunk(ch, False), 0)
        for _ in range(n_out_dma):
            pltpu.make_async_copy(
                embf_v.at[pl.ds(0, 8), pl.ds(0, BW)],
                out_hbm.at[0, 0, wid], sem_o).wait()

    return pl.kernel(
        body,
        out_type=jax.ShapeDtypeStruct((L, NC1, NB1, 8, BW), jnp.float32),
        mesh=plsc.VectorSubcoreMesh(core_axis_name="c", subcore_axis_name="s",
                                    num_cores=NC, num_subcores=NS),
        compiler_params=pltpu.CompilerParams(use_tc_tiling_on_sc=False,
                                             needs_layout_passes=False),
        scratch_types=[
            pltpu.VMEM((PER_W,), jnp.int32),
            pltpu.VMEM((PER_W,), jnp.int32),
            pltpu.VMEM((PER_W,), jnp.int32),
            pltpu.VMEM((C,), jnp.int32),
            pltpu.VMEM((C,), jnp.int32),
            pltpu.VMEM((C,), jnp.int32),
            pltpu.VMEM((C, ND), jnp.float32),
            pltpu.VMEM((ND, C), jnp.float32),
            pltpu.VMEM((16, C), jnp.float32),
            pltpu.VMEM((12,), jnp.float32),
            pltpu.SemaphoreType.DMA,
            pltpu.SemaphoreType.DMA,
            pltpu.SemaphoreType.DMA,
            pltpu.SemaphoreType.DMA,
            pltpu.SemaphoreType.DMA,
        ],
    )


def kernel(idx, joint_info, embedding, edge_idx_map, edge_emb_weight, bond_type):
    B, L = idx.shape
    N = B * L
    V, ND = embedding.shape
    MJ = edge_idx_map.shape[1]
    E, EW = edge_emb_weight.shape
    ED = EW + 3
    idx_f = idx.reshape(N)
    jp_f = joint_info[..., 0].reshape(N)
    bb_f = joint_info[..., 1].reshape(N)
    emap_t = edge_idx_map.T.reshape(MJ * V)
    ewt = edge_emb_weight.T
    ew_cols = [ewt[c] for c in range(EW)]
    bt_f = bond_type.reshape(-1)
    out5 = _make_sc_call(N, B, L, V, MJ, ND, ED, E)(
        idx_f, jp_f, bb_f, embedding, emap_t, *ew_cols, bt_f)
    return out5.transpose(2, 4, 0, 1, 3).reshape(B, L, ND + ED)
